# Initial kernel scaffold; baseline (speedup 1.0000x reference)
import jax
import jax.numpy as jnp
from jax.experimental import pallas as pl


def kernel(feat_p2, feat_p3, feat_p4, feat_p5, proposals, im_h, im_w):
    raise NotImplementedError("write your pallas kernel here")


# speedup vs baseline: 5.5761x; 5.5761x over previous
import functools
import jax
import jax.numpy as jnp
from jax import lax
from jax.experimental import pallas as pl
from jax.experimental.pallas import tpu as pltpu, tpu_sc as plsc

OH, OW, SR = 7, 7, 2
NBINS = OH * OW
K = SR * SR * 4
ROWS = NBINS * K
GROUP_BINS = 7
GROUP_ROWS = GROUP_BINS * K
NGROUPS = NBINS // GROUP_BINS
C = 256
NCHUNK = C // 16

HS = (256, 128, 64, 32)
BASES = (0, 65536, 81920, 86016)
TABLE_ROWS = 87040

NP_PAD = 1024
NWORKERS = 32
PPW = NP_PAD // NWORKERS


def _build_idx_wt(proposals):
    N = proposals.shape[0]
    areas = (proposals[:, 2] - proposals[:, 0]) * (proposals[:, 3] - proposals[:, 1])
    scale = jnp.sqrt(areas)
    levels = jnp.clip(jnp.floor(jnp.log2(scale / 224.0) + 4.0).astype(jnp.int32), 2, 5)
    li = levels - 2
    Ls = jnp.asarray(HS, jnp.float32)[li]
    base = jnp.asarray(BASES, jnp.int32)[li]
    sp = 1.0 / jnp.asarray([4.0, 8.0, 16.0, 32.0], jnp.float32)[li]

    x1 = proposals[:, 0] * sp
    y1 = proposals[:, 1] * sp
    x2 = proposals[:, 2] * sp
    y2 = proposals[:, 3] * sp
    bin_w = jnp.maximum(x2 - x1, 1.0) / OW
    bin_h = jnp.maximum(y2 - y1, 1.0) / OH
    ty = (jnp.arange(OH * SR, dtype=jnp.float32) + 0.5) / SR
    ys = y1[:, None] + ty[None, :] * bin_h[:, None]
    xs = x1[:, None] + ty[None, :] * bin_w[:, None]

    def axis(ss):
        v = (ss >= -1.0) & (ss <= Ls[:, None])
        sc = jnp.clip(ss, 0.0, Ls[:, None] - 1.0)
        i0 = jnp.floor(sc).astype(jnp.int32)
        i1 = jnp.minimum(i0 + 1, Ls[:, None].astype(jnp.int32) - 1)
        frac = sc - i0.astype(jnp.float32)
        w0 = jnp.where(v, 1.0 - frac, 0.0)
        w1 = jnp.where(v, frac, 0.0)
        return i0, i1, w0, w1

    y0, y1i, wy0, wy1 = axis(ys)
    x0, x1i, wx0, wx1 = axis(xs)

    Wsi = Ls.astype(jnp.int32)
    yi = jnp.stack([y0, y0, y1i, y1i], axis=-1)
    wy = jnp.stack([wy0, wy0, wy1, wy1], axis=-1)
    xi = jnp.stack([x0, x1i, x0, x1i], axis=-1)
    wx = jnp.stack([wx0, wx1, wx0, wx1], axis=-1)
    idx = (base[:, None, None, None] + yi[:, :, None, :] * Wsi[:, None, None, None]
           + xi[:, None, :, :])
    wt = (wy[:, :, None, :] * wx[:, None, :, :]) * 0.25
    idx = idx.reshape(N, OH, SR, OW, SR, 4).transpose(0, 1, 3, 2, 4, 5).reshape(N, NBINS, K)
    wt = wt.reshape(N, OH, SR, OW, SR, 4).transpose(0, 1, 3, 2, 4, 5).reshape(N, NBINS, K)
    return idx.reshape(N, ROWS), wt.reshape(N, ROWS)


def _sc_body(table, idxs, wts, out, idx_g, wt_v, rows, out_v, sem):
    wid = lax.axis_index("s") * 2 + lax.axis_index("c")

    def prop_body(i, carry):
        gp = wid * PPW + i
        pltpu.sync_copy(wts.at[pl.ds(pl.multiple_of(gp * ROWS, 8), ROWS)], wt_v)
        for g in range(NGROUPS):
            pltpu.sync_copy(
                idxs.at[pl.ds(pl.multiple_of(gp * ROWS + g * GROUP_ROWS, 8),
                              GROUP_ROWS)], idx_g)
            pltpu.async_copy(table.at[idx_g], rows, sem).wait()

            def bin_body(b, _):
                bb = g * GROUP_BINS + b
                wv = wt_v[pl.ds(bb * K, K)]
                acc = [jnp.zeros((16,), jnp.float32) for _ in range(NCHUNK)]
                for k in range(K):
                    w = jnp.full((16,), wv[k], jnp.float32)
                    for c in range(NCHUNK):
                        acc[c] = acc[c] + w * rows[b * K + k, pl.ds(c * 16, 16)]
                for c in range(NCHUNK):
                    out_v[pl.ds(bb * C + c * 16, 16)] = acc[c]
                return 0

            lax.fori_loop(0, GROUP_BINS, bin_body, 0)
        pltpu.sync_copy(out_v, out.at[pl.ds(pl.multiple_of(gp * NBINS * C, 8),
                                            NBINS * C)])
        return 0

    lax.fori_loop(0, PPW, prop_body, 0)


@jax.jit
def _run(table, idx_p, wt_p):
    mesh = plsc.VectorSubcoreMesh(core_axis_name="c", subcore_axis_name="s")
    return pl.kernel(
        _sc_body,
        out_type=jax.ShapeDtypeStruct((NP_PAD * NBINS * C,), jnp.float32),
        mesh=mesh,
        scratch_types=[
            pltpu.VMEM((GROUP_ROWS,), jnp.int32),
            pltpu.VMEM((ROWS,), jnp.float32),
            pltpu.VMEM((GROUP_ROWS, C), jnp.float32),
            pltpu.VMEM((NBINS * C,), jnp.float32),
            pltpu.SemaphoreType.DMA,
        ],
    )(table, idx_p, wt_p)


def kernel(feat_p2, feat_p3, feat_p4, feat_p5, proposals, im_h, im_w):
    N = proposals.shape[0]
    table = jnp.concatenate(
        [jnp.transpose(f[0], (1, 2, 0)).reshape(-1, C)
         for f in (feat_p2, feat_p3, feat_p4, feat_p5)], axis=0)
    idx, wt = _build_idx_wt(proposals)
    idx_p = jnp.zeros((NP_PAD, ROWS), jnp.int32).at[:N].set(idx).reshape(-1)
    wt_p = jnp.zeros((NP_PAD, ROWS), jnp.float32).at[:N].set(wt).reshape(-1)
    out = _run(table, idx_p, wt_p)
    return out.reshape(NP_PAD, OH, OW, C)[:N].transpose(0, 3, 1, 2)


# speedup vs baseline: 6.7432x; 1.2093x over previous
import functools
import jax
import jax.numpy as jnp
from jax import lax
from jax.experimental import pallas as pl
from jax.experimental.pallas import tpu as pltpu, tpu_sc as plsc

OH, OW, SR = 7, 7, 2
NBINS = OH * OW
K = SR * SR * 4
ROWS = NBINS * K
GROUP_BINS = 7
GROUP_ROWS = GROUP_BINS * K
NGROUPS = NBINS // GROUP_BINS
C = 256
NCHUNK = C // 16

HS = (256, 128, 64, 32)
BASES = (0, 65536, 81920, 86016)
TABLE_ROWS = 87040

NP_PAD = 1024
NWORKERS = 32
PPW = NP_PAD // NWORKERS


def _build_idx_wt(proposals):
    N = proposals.shape[0]
    areas = (proposals[:, 2] - proposals[:, 0]) * (proposals[:, 3] - proposals[:, 1])
    scale = jnp.sqrt(areas)
    levels = jnp.clip(jnp.floor(jnp.log2(scale / 224.0) + 4.0).astype(jnp.int32), 2, 5)
    li = levels - 2
    Ls = jnp.asarray(HS, jnp.float32)[li]
    base = jnp.asarray(BASES, jnp.int32)[li]
    sp = 1.0 / jnp.asarray([4.0, 8.0, 16.0, 32.0], jnp.float32)[li]

    x1 = proposals[:, 0] * sp
    y1 = proposals[:, 1] * sp
    x2 = proposals[:, 2] * sp
    y2 = proposals[:, 3] * sp
    bin_w = jnp.maximum(x2 - x1, 1.0) / OW
    bin_h = jnp.maximum(y2 - y1, 1.0) / OH
    ty = (jnp.arange(OH * SR, dtype=jnp.float32) + 0.5) / SR
    ys = y1[:, None] + ty[None, :] * bin_h[:, None]
    xs = x1[:, None] + ty[None, :] * bin_w[:, None]

    def axis(ss):
        v = (ss >= -1.0) & (ss <= Ls[:, None])
        sc = jnp.clip(ss, 0.0, Ls[:, None] - 1.0)
        i0 = jnp.floor(sc).astype(jnp.int32)
        i1 = jnp.minimum(i0 + 1, Ls[:, None].astype(jnp.int32) - 1)
        frac = sc - i0.astype(jnp.float32)
        w0 = jnp.where(v, 1.0 - frac, 0.0)
        w1 = jnp.where(v, frac, 0.0)
        return i0, i1, w0, w1

    y0, y1i, wy0, wy1 = axis(ys)
    x0, x1i, wx0, wx1 = axis(xs)

    Wsi = Ls.astype(jnp.int32)
    yi = jnp.stack([y0, y0, y1i, y1i], axis=-1)
    wy = jnp.stack([wy0, wy0, wy1, wy1], axis=-1)
    xi = jnp.stack([x0, x1i, x0, x1i], axis=-1)
    wx = jnp.stack([wx0, wx1, wx0, wx1], axis=-1)
    idx = (base[:, None, None, None] + yi[:, :, None, :] * Wsi[:, None, None, None]
           + xi[:, None, :, :])
    wt = (wy[:, :, None, :] * wx[:, None, :, :]) * 0.25
    idx = idx.reshape(N, OH, SR, OW, SR, 4).transpose(0, 1, 3, 2, 4, 5).reshape(N, NBINS, K)
    wt = wt.reshape(N, OH, SR, OW, SR, 4).transpose(0, 1, 3, 2, 4, 5).reshape(N, NBINS, K)
    return idx.reshape(N, ROWS), wt.reshape(N, ROWS)


def _sc_body(table, idxs, wts, out, idx_v, wt_v, rows0, rows1, out_v,
             sem_i, sem0, sem1, sem_o):
    wid = lax.axis_index("s") * 2 + lax.axis_index("c")
    bufs = (rows0, rows1)
    sems = (sem0, sem1)

    def prop_body(i, carry):
        gp = wid * PPW + i
        off = pl.multiple_of(gp * ROWS, 8)
        ci = pltpu.async_copy(idxs.at[pl.ds(off, ROWS)], idx_v, sem_i)
        cw = pltpu.async_copy(wts.at[pl.ds(off, ROWS)], wt_v, sem_i)
        ci.wait()
        cw.wait()
        cps = [None] * NGROUPS
        for g in range(2):
            cps[g] = pltpu.async_copy(
                table.at[idx_v.at[pl.ds(g * GROUP_ROWS, GROUP_ROWS)]],
                bufs[g], sems[g])

        @pl.when(i > 0)
        def _():
            pltpu.make_async_copy(
                out_v, out.at[pl.ds(pl.multiple_of(gp * NBINS * C, 8),
                                    NBINS * C)], sem_o).wait()

        for g in range(NGROUPS):
            cps[g].wait()
            rows = bufs[g % 2]

            def bin_body(b, _):
                bb = g * GROUP_BINS + b
                wv = wt_v[pl.ds(bb * K, K)]
                acc = [jnp.zeros((16,), jnp.float32) for _ in range(NCHUNK)]
                for k in range(K):
                    w = jnp.full((16,), wv[k], jnp.float32)
                    for c in range(NCHUNK):
                        acc[c] = acc[c] + w * rows[b * K + k, pl.ds(c * 16, 16)]
                for c in range(NCHUNK):
                    out_v[pl.ds(bb * C + c * 16, 16)] = acc[c]
                return 0

            lax.fori_loop(0, GROUP_BINS, bin_body, 0)
            if g + 2 < NGROUPS:
                cps[g + 2] = pltpu.async_copy(
                    table.at[idx_v.at[pl.ds((g + 2) * GROUP_ROWS, GROUP_ROWS)]],
                    rows, sems[g % 2])

        pltpu.async_copy(
            out_v, out.at[pl.ds(pl.multiple_of(gp * NBINS * C, 8), NBINS * C)],
            sem_o)
        return 0

    lax.fori_loop(0, PPW, prop_body, 0)
    last = (wid * PPW + PPW - 1) * NBINS * C
    pltpu.make_async_copy(
        out_v, out.at[pl.ds(pl.multiple_of(last, 8), NBINS * C)], sem_o).wait()


@jax.jit
def _run(table, idx_p, wt_p):
    mesh = plsc.VectorSubcoreMesh(core_axis_name="c", subcore_axis_name="s")
    return pl.kernel(
        _sc_body,
        out_type=jax.ShapeDtypeStruct((NP_PAD * NBINS * C,), jnp.float32),
        mesh=mesh,
        scratch_types=[
            pltpu.VMEM((ROWS,), jnp.int32),
            pltpu.VMEM((ROWS,), jnp.float32),
            pltpu.VMEM((GROUP_ROWS, C), jnp.float32),
            pltpu.VMEM((GROUP_ROWS, C), jnp.float32),
            pltpu.VMEM((NBINS * C,), jnp.float32),
            pltpu.SemaphoreType.DMA,
            pltpu.SemaphoreType.DMA,
            pltpu.SemaphoreType.DMA,
            pltpu.SemaphoreType.DMA,
        ],
    )(table, idx_p, wt_p)


def kernel(feat_p2, feat_p3, feat_p4, feat_p5, proposals, im_h, im_w):
    N = proposals.shape[0]
    table = jnp.concatenate(
        [jnp.transpose(f[0], (1, 2, 0)).reshape(-1, C)
         for f in (feat_p2, feat_p3, feat_p4, feat_p5)], axis=0)
    idx, wt = _build_idx_wt(proposals)
    idx_p = jnp.zeros((NP_PAD, ROWS), jnp.int32).at[:N].set(idx).reshape(-1)
    wt_p = jnp.zeros((NP_PAD, ROWS), jnp.float32).at[:N].set(wt).reshape(-1)
    out = _run(table, idx_p, wt_p)
    return out.reshape(NP_PAD, OH, OW, C)[:N].transpose(0, 3, 1, 2)
